# baseline (speedup 1.0000x reference)
import functools

import jax
import jax.numpy as jnp
from jax import lax
from jax.experimental import pallas as pl
from jax.experimental.pallas import tpu as pltpu
from jax.experimental.pallas import tpu_sc as plsc

_INT_MIN = -2147483648

_N = 1048576
_NW = 32
_CHUNK = _N // _NW
_VPT = _CHUNK // 16
_NB1 = 2048
_NB2 = 2048
_NB3 = 1024
_R1 = int(0.05 * (_N - 1))
_F1 = 0.05 * (_N - 1) - _R1
_R2 = int(0.95 * (_N - 1))
_F2 = 0.95 * (_N - 1) - _R2
_ALPHA = 0.01


def _sc_skey(b):
    return jnp.where(b >= 0, b, jnp.int32(_INT_MIN) - b)


def _skey_to_float(k):
    return lax.bitcast_convert_type(
        jnp.where(k >= 0, k, jnp.int32(_INT_MIN) - k), jnp.float32)


def _b1_of(sk):
    return lax.bitwise_xor(
        lax.bitwise_and(lax.shift_right_arithmetic(sk, 21), jnp.int32(0x7FF)),
        jnp.int32(0x400))


def _b2_of(sk):
    return lax.bitwise_and(lax.shift_right_arithmetic(sk, 10), jnp.int32(0x7FF))


def _sc_wid():
    return lax.axis_index("s") * 2 + lax.axis_index("c")


def _zero_vmem(ref, nwords):
    z = jnp.zeros((16,), jnp.int32)

    def zbody(i, c):
        ref[pl.ds(i * 16, 16)] = z
        return c

    lax.fori_loop(0, nwords // 16, zbody, 0)


def _sc1_body(x_hbm, out_hbm, data_v, hist_v):
    wid = _sc_wid()
    pltpu.sync_copy(x_hbm.at[pl.ds(wid * _CHUNK, _CHUNK)], data_v)
    _zero_vmem(hist_v, 16 * _NB1)
    laneoff = jnp.arange(16, dtype=jnp.int32) * _NB1
    ones = jnp.ones((16,), jnp.int32)

    def body(i, c):
        sk = _sc_skey(data_v[pl.ds(i * 16, 16)])
        plsc.addupdate_scatter(hist_v, [laneoff + _b1_of(sk)], ones)
        return c

    lax.fori_loop(0, _VPT, body, 0)
    pltpu.sync_copy(hist_v, out_hbm.at[wid])


def _sc2_body(x_hbm, params_hbm, out_hbm, data_v, hist_v, plo_v, phi_v):
    wid = _sc_wid()
    pltpu.sync_copy(params_hbm.at[pl.ds(0, 16)], plo_v)
    pltpu.sync_copy(params_hbm.at[pl.ds(128, 16)], phi_v)
    pltpu.sync_copy(x_hbm.at[pl.ds(wid * _CHUNK, _CHUNK)], data_v)
    _zero_vmem(hist_v, 2 * 16 * _NB2)
    laneoff = jnp.arange(16, dtype=jnp.int32) * _NB2
    ones = jnp.ones((16,), jnp.int32)
    plo = plo_v[...]
    phi = phi_v[...]

    def body(i, c):
        sk = _sc_skey(data_v[pl.ds(i * 16, 16)])
        b1 = _b1_of(sk)
        idx = laneoff + _b2_of(sk)
        plsc.addupdate_scatter(hist_v, [idx], ones, mask=b1 == plo)
        plsc.addupdate_scatter(hist_v, [idx + 16 * _NB2], ones, mask=b1 == phi)
        return c

    lax.fori_loop(0, _VPT, body, 0)
    pltpu.sync_copy(hist_v.at[pl.ds(0, 16 * _NB2)], out_hbm.at[0, wid])
    pltpu.sync_copy(hist_v.at[pl.ds(16 * _NB2, 16 * _NB2)], out_hbm.at[1, wid])


def _sc3_body(x_hbm, params_hbm, out_hbm, minab_hbm, data_v, hist_v, p_v,
              m_v):
    wid = _sc_wid()
    for r in range(6):
        pltpu.sync_copy(params_hbm.at[pl.ds(r * 128, 16)],
                        p_v.at[pl.ds(r * 16, 16)])
    pltpu.sync_copy(x_hbm.at[pl.ds(wid * _CHUNK, _CHUNK)], data_v)
    _zero_vmem(hist_v, 2 * 16 * _NB3)
    laneoff = jnp.arange(16, dtype=jnp.int32) * _NB3
    ones = jnp.ones((16,), jnp.int32)
    b1lo = p_v[pl.ds(0, 16)]
    b1hi = p_v[pl.ds(16, 16)]
    b2lo = p_v[pl.ds(32, 16)]
    b2hi = p_v[pl.ds(48, 16)]
    tslo = p_v[pl.ds(64, 16)]
    tshi = p_v[pl.ds(80, 16)]
    imax = jnp.full((16,), 2147483647, jnp.int32)

    def body(i, carry):
        mlo, mhi = carry
        sk = _sc_skey(data_v[pl.ds(i * 16, 16)])
        b1 = _b1_of(sk)
        b2 = _b2_of(sk)
        idx = laneoff + lax.bitwise_and(sk, jnp.int32(0x3FF))
        plsc.addupdate_scatter(hist_v, [idx], ones,
                               mask=(b1 == b1lo) & (b2 == b2lo))
        plsc.addupdate_scatter(hist_v, [idx + 16 * _NB3], ones,
                               mask=(b1 == b1hi) & (b2 == b2hi))
        mlo = jnp.minimum(mlo, jnp.where(sk >= tslo, sk, imax))
        mhi = jnp.minimum(mhi, jnp.where(sk >= tshi, sk, imax))
        return mlo, mhi

    mlo, mhi = lax.fori_loop(0, _VPT, body, (imax, imax))
    m_v[pl.ds(0, 16)] = mlo
    m_v[pl.ds(16, 16)] = mhi
    pltpu.sync_copy(hist_v.at[pl.ds(0, 16 * _NB3)], out_hbm.at[0, wid])
    pltpu.sync_copy(hist_v.at[pl.ds(16 * _NB3, 16 * _NB3)], out_hbm.at[1, wid])
    pltpu.sync_copy(m_v.at[pl.ds(0, 16)], minab_hbm.at[pl.ds(wid * 16, 16)])
    pltpu.sync_copy(m_v.at[pl.ds(16, 16)],
                    minab_hbm.at[pl.ds(512 + wid * 16, 16)])


def _bucket_search(g, iota, rank, nbits):
    b = jnp.int32(0)
    for k in range(nbits - 1, -1, -1):
        cand = b + jnp.int32(1 << k)
        s = jnp.sum(jnp.where(iota < cand, g, 0))
        b = jnp.where(s <= rank, cand, b)
    below = jnp.sum(jnp.where(iota < b, g, 0))
    return b, below


def _tc1_kernel(h_ref, out_ref):
    g = jnp.sum(h_ref[...], axis=0, keepdims=True)
    iota = lax.broadcasted_iota(jnp.int32, (1, _NB1), 1)
    b1lo, cblo = _bucket_search(g, iota, jnp.int32(_R1), 11)
    b1hi, cbhi = _bucket_search(g, iota, jnp.int32(_R2), 11)
    for r, v in enumerate([b1lo, b1hi, cblo, cbhi]):
        out_ref[r:r + 1, :] = jnp.full((1, 128), v, jnp.int32)
    for r in range(4, 8):
        out_ref[r:r + 1, :] = jnp.zeros((1, 128), jnp.int32)


def _tc2_kernel(h_ref, params_ref, out_ref):
    g_lo = jnp.sum(h_ref[0:512], axis=0, keepdims=True)
    g_hi = jnp.sum(h_ref[512:1024], axis=0, keepdims=True)
    iota = lax.broadcasted_iota(jnp.int32, (1, _NB2), 1)
    b1lo = params_ref[0, 0]
    b1hi = params_ref[1, 0]
    cblo = params_ref[2, 0]
    cbhi = params_ref[3, 0]
    b2lo, wlo = _bucket_search(g_lo, iota, jnp.int32(_R1) - cblo, 11)
    b2hi, whi = _bucket_search(g_hi, iota, jnp.int32(_R2) - cbhi, 11)
    cb2lo = cblo + wlo
    cb2hi = cbhi + whi
    tslo = lax.bitwise_xor(
        lax.shift_left((b1lo << 11 | b2lo) + 1, 10), jnp.int32(_INT_MIN))
    tshi = lax.bitwise_xor(
        lax.shift_left((b1hi << 11 | b2hi) + 1, 10), jnp.int32(_INT_MIN))
    for r, v in enumerate([b1lo, b1hi, b2lo, b2hi, tslo, tshi, cb2lo, cb2hi]):
        out_ref[r:r + 1, :] = jnp.full((1, 128), v, jnp.int32)


def _tc3_kernel(h_ref, minab_ref, params_ref, ema_ref, out0_ref, out1_ref):
    iota = lax.broadcasted_iota(jnp.int32, (1, _NB3), 1)

    def resolve(rows, rank, frac, b1, b2, cb2, mrow):
        g = jnp.sum(h_ref[rows], axis=0, keepdims=True)
        b3, below = _bucket_search(g, iota, rank - cb2, 10)
        c_at = jnp.sum(jnp.where(iota == b3, g, 0))
        cnt_le = cb2 + below + c_at
        base = lax.bitwise_xor(
            lax.shift_left(b1 << 11 | b2, 10), jnp.int32(_INT_MIN))
        k1 = base | b3
        j2 = jnp.min(jnp.where((iota > b3) & (g > 0), iota, jnp.int32(_NB3)))
        k2_far = jnp.min(minab_ref[mrow:mrow + 1, :])
        k2 = jnp.where(cnt_le >= rank + 2, k1,
                       jnp.where(j2 < _NB3, base | j2, k2_far))
        v1 = _skey_to_float(k1)
        v2 = _skey_to_float(k2)
        return v1 + jnp.float32(frac) * (v2 - v1)

    q_lo = resolve(slice(0, 512), jnp.int32(_R1), _F1, params_ref[0, 0],
                   params_ref[2, 0], params_ref[6, 0], 0)
    q_hi = resolve(slice(512, 1024), jnp.int32(_R2), _F2, params_ref[1, 0],
                   params_ref[3, 0], params_ref[7, 0], 1)
    new0 = jnp.float32(_ALPHA) * q_lo + jnp.float32(1 - _ALPHA) * ema_ref[0]
    new1 = jnp.float32(_ALPHA) * q_hi + jnp.float32(1 - _ALPHA) * ema_ref[1]
    out0_ref[...] = jnp.full((1, 128), new0, jnp.float32)
    out1_ref[...] = jnp.full((1, 128), jnp.maximum(new1 - new0, 1.0),
                             jnp.float32)


def kernel(x, ema_vals):
    xf = lax.bitcast_convert_type(x, jnp.int32).reshape(-1)
    mesh = plsc.VectorSubcoreMesh(core_axis_name="c", subcore_axis_name="s",
                                  num_cores=2, num_subcores=16)
    i32 = jnp.int32

    h1 = pl.kernel(
        _sc1_body,
        out_type=jax.ShapeDtypeStruct((_NW, 16 * _NB1), i32),
        mesh=mesh,
        compiler_params=pltpu.CompilerParams(needs_layout_passes=False),
        scratch_types=[pltpu.VMEM((_CHUNK,), i32),
                       pltpu.VMEM((16 * _NB1,), i32)],
    )(xf)

    params1 = pl.pallas_call(
        _tc1_kernel,
        out_shape=jax.ShapeDtypeStruct((8, 128), i32),
        in_specs=[pl.BlockSpec(memory_space=pltpu.VMEM)],
        out_specs=pl.BlockSpec(memory_space=pltpu.VMEM),
    )(h1.reshape(_NW * 16, _NB1))

    h2 = pl.kernel(
        _sc2_body,
        out_type=jax.ShapeDtypeStruct((2, _NW, 16 * _NB2), i32),
        mesh=mesh,
        compiler_params=pltpu.CompilerParams(needs_layout_passes=False),
        scratch_types=[pltpu.VMEM((_CHUNK,), i32),
                       pltpu.VMEM((2 * 16 * _NB2,), i32),
                       pltpu.VMEM((16,), i32),
                       pltpu.VMEM((16,), i32)],
    )(xf, params1.reshape(-1))

    params2 = pl.pallas_call(
        _tc2_kernel,
        out_shape=jax.ShapeDtypeStruct((8, 128), i32),
        in_specs=[pl.BlockSpec(memory_space=pltpu.VMEM),
                  pl.BlockSpec(memory_space=pltpu.SMEM)],
        out_specs=pl.BlockSpec(memory_space=pltpu.VMEM),
    )(h2.reshape(2 * _NW * 16, _NB2), params1)

    h3, minab = pl.kernel(
        _sc3_body,
        out_type=(jax.ShapeDtypeStruct((2, _NW, 16 * _NB3), i32),
                  jax.ShapeDtypeStruct((2 * _NW * 16,), i32)),
        mesh=mesh,
        compiler_params=pltpu.CompilerParams(needs_layout_passes=False),
        scratch_types=[pltpu.VMEM((_CHUNK,), i32),
                       pltpu.VMEM((2 * 16 * _NB3,), i32),
                       pltpu.VMEM((6 * 16,), i32),
                       pltpu.VMEM((2 * 16,), i32)],
    )(xf, params2.reshape(-1))

    out0, out1 = pl.pallas_call(
        _tc3_kernel,
        out_shape=(jax.ShapeDtypeStruct((1, 128), jnp.float32),
                   jax.ShapeDtypeStruct((1, 128), jnp.float32)),
        in_specs=[pl.BlockSpec(memory_space=pltpu.VMEM),
                  pl.BlockSpec(memory_space=pltpu.VMEM),
                  pl.BlockSpec(memory_space=pltpu.SMEM),
                  pl.BlockSpec(memory_space=pltpu.SMEM)],
        out_specs=(pl.BlockSpec(memory_space=pltpu.VMEM),
                   pl.BlockSpec(memory_space=pltpu.VMEM)),
    )(h3.reshape(2 * _NW * 16, _NB3), minab.reshape(2, _NW * 16), params2,
      ema_vals)
    return out0[0, 0], out1[0, 0]


# speedup vs baseline: 1.4478x; 1.4478x over previous
import functools

import jax
import jax.numpy as jnp
from jax import lax
from jax.experimental import pallas as pl
from jax.experimental.pallas import tpu as pltpu
from jax.experimental.pallas import tpu_sc as plsc

_INT_MIN = -2147483648

_N = 1048576
_NW = 32
_CHUNK = _N // _NW
_VPT = _CHUNK // 16
_NB1 = 2048
_NB2 = 2048
_NB3 = 1024
_R1 = int(0.05 * (_N - 1))
_F1 = 0.05 * (_N - 1) - _R1
_R2 = int(0.95 * (_N - 1))
_F2 = 0.95 * (_N - 1) - _R2
_ALPHA = 0.01


def _sc_skey(b):
    return jnp.where(b >= 0, b, jnp.int32(_INT_MIN) - b)


def _skey_to_float(k):
    return lax.bitcast_convert_type(
        jnp.where(k >= 0, k, jnp.int32(_INT_MIN) - k), jnp.float32)


def _b1_of(sk):
    return lax.bitwise_xor(
        lax.bitwise_and(lax.shift_right_arithmetic(sk, 21), jnp.int32(0x7FF)),
        jnp.int32(0x400))


def _b2_of(sk):
    return lax.bitwise_and(lax.shift_right_arithmetic(sk, 10), jnp.int32(0x7FF))


def _sc_wid():
    return lax.axis_index("s") * 2 + lax.axis_index("c")


def _zero_vmem(ref, nwords):
    z = jnp.zeros((16,), jnp.int32)

    def zbody(i, c):
        for u in range(8):
            ref[pl.ds((i * 8 + u) * 16, 16)] = z
        return c

    lax.fori_loop(0, nwords // 128, zbody, 0)


def _lane_reduce(hist_v, g_v, nb):
    def rbody(j, c):
        acc = hist_v[pl.ds(j * 16, 16)]
        for l in range(1, 16):
            acc = acc + hist_v[pl.ds(l * nb + j * 16, 16)]
        g_v[pl.ds(j * 16, 16)] = acc
        return c

    lax.fori_loop(0, nb // 16, rbody, 0)


def _sc1_body(x_hbm, out_hbm, data_v, hist_v, g_v):
    wid = _sc_wid()
    pltpu.sync_copy(x_hbm.at[pl.ds(wid * _CHUNK, _CHUNK)], data_v)
    _zero_vmem(hist_v, 16 * _NB1)
    laneoff = jnp.arange(16, dtype=jnp.int32) * _NB1
    ones = jnp.ones((16,), jnp.int32)

    def body(i, c):
        for u in range(8):
            sk = _sc_skey(data_v[pl.ds((i * 8 + u) * 16, 16)])
            plsc.addupdate_scatter(hist_v, [laneoff + _b1_of(sk)], ones)
        return c

    lax.fori_loop(0, _VPT // 8, body, 0)
    _lane_reduce(hist_v, g_v, _NB1)
    pltpu.sync_copy(g_v, out_hbm.at[wid])


def _sc2_body(x_hbm, params_hbm, out_hbm, data_v, hist_v, g_v, plo_v, phi_v):
    wid = _sc_wid()
    pltpu.sync_copy(params_hbm.at[pl.ds(0, 16)], plo_v)
    pltpu.sync_copy(params_hbm.at[pl.ds(128, 16)], phi_v)
    pltpu.sync_copy(x_hbm.at[pl.ds(wid * _CHUNK, _CHUNK)], data_v)
    _zero_vmem(hist_v, 2 * 16 * _NB2)
    laneoff = jnp.arange(16, dtype=jnp.int32) * _NB2
    ones = jnp.ones((16,), jnp.int32)
    plo = plo_v[...]
    phi = phi_v[...]

    def body(i, c):
        for u in range(8):
            sk = _sc_skey(data_v[pl.ds((i * 8 + u) * 16, 16)])
            b1 = _b1_of(sk)
            idx = laneoff + _b2_of(sk)
            plsc.addupdate_scatter(hist_v, [idx], ones, mask=b1 == plo)
            plsc.addupdate_scatter(hist_v, [idx + 16 * _NB2], ones,
                                   mask=b1 == phi)
        return c

    lax.fori_loop(0, _VPT // 8, body, 0)
    _lane_reduce(hist_v, g_v, _NB2)
    _lane_reduce(hist_v.at[pl.ds(16 * _NB2, 16 * _NB2)],
                 g_v.at[pl.ds(_NB2, _NB2)], _NB2)
    pltpu.sync_copy(g_v.at[pl.ds(0, _NB2)], out_hbm.at[0, wid])
    pltpu.sync_copy(g_v.at[pl.ds(_NB2, _NB2)], out_hbm.at[1, wid])


def _sc3_body(x_hbm, params_hbm, out_hbm, minab_hbm, data_v, hist_v, g_v, p_v,
              m_v):
    wid = _sc_wid()
    for r in range(6):
        pltpu.sync_copy(params_hbm.at[pl.ds(r * 128, 16)],
                        p_v.at[pl.ds(r * 16, 16)])
    pltpu.sync_copy(x_hbm.at[pl.ds(wid * _CHUNK, _CHUNK)], data_v)
    _zero_vmem(hist_v, 2 * 16 * _NB3)
    laneoff = jnp.arange(16, dtype=jnp.int32) * _NB3
    ones = jnp.ones((16,), jnp.int32)
    b1lo = p_v[pl.ds(0, 16)]
    b1hi = p_v[pl.ds(16, 16)]
    b2lo = p_v[pl.ds(32, 16)]
    b2hi = p_v[pl.ds(48, 16)]
    tslo = p_v[pl.ds(64, 16)]
    tshi = p_v[pl.ds(80, 16)]
    imax = jnp.full((16,), 2147483647, jnp.int32)

    def body(i, carry):
        mlo, mhi = carry
        clo = []
        chi = []
        for u in range(8):
            sk = _sc_skey(data_v[pl.ds((i * 8 + u) * 16, 16)])
            b1 = _b1_of(sk)
            b2 = _b2_of(sk)
            idx = laneoff + lax.bitwise_and(sk, jnp.int32(0x3FF))
            plsc.addupdate_scatter(hist_v, [idx], ones,
                                   mask=(b1 == b1lo) & (b2 == b2lo))
            plsc.addupdate_scatter(hist_v, [idx + 16 * _NB3], ones,
                                   mask=(b1 == b1hi) & (b2 == b2hi))
            clo.append(jnp.where(sk >= tslo, sk, imax))
            chi.append(jnp.where(sk >= tshi, sk, imax))
        while len(clo) > 1:
            clo = [jnp.minimum(a, b) for a, b in zip(clo[::2], clo[1::2])]
            chi = [jnp.minimum(a, b) for a, b in zip(chi[::2], chi[1::2])]
        return jnp.minimum(mlo, clo[0]), jnp.minimum(mhi, chi[0])

    mlo, mhi = lax.fori_loop(0, _VPT // 8, body, (imax, imax))
    m_v[pl.ds(0, 16)] = mlo
    m_v[pl.ds(16, 16)] = mhi
    _lane_reduce(hist_v, g_v, _NB3)
    _lane_reduce(hist_v.at[pl.ds(16 * _NB3, 16 * _NB3)],
                 g_v.at[pl.ds(_NB3, _NB3)], _NB3)
    pltpu.sync_copy(g_v.at[pl.ds(0, _NB3)], out_hbm.at[0, wid])
    pltpu.sync_copy(g_v.at[pl.ds(_NB3, _NB3)], out_hbm.at[1, wid])
    pltpu.sync_copy(m_v.at[pl.ds(0, 16)], minab_hbm.at[pl.ds(wid * 16, 16)])
    pltpu.sync_copy(m_v.at[pl.ds(16, 16)],
                    minab_hbm.at[pl.ds(512 + wid * 16, 16)])


def _bucket_search(g, iota, rank, nbits):
    b = jnp.int32(0)
    for k in range(nbits - 1, -1, -1):
        cand = b + jnp.int32(1 << k)
        s = jnp.sum(jnp.where(iota < cand, g, 0))
        b = jnp.where(s <= rank, cand, b)
    below = jnp.sum(jnp.where(iota < b, g, 0))
    return b, below


def _tc1_kernel(h_ref, out_ref):
    g = jnp.sum(h_ref[...], axis=0, keepdims=True)
    iota = lax.broadcasted_iota(jnp.int32, (1, _NB1), 1)
    b1lo, cblo = _bucket_search(g, iota, jnp.int32(_R1), 11)
    b1hi, cbhi = _bucket_search(g, iota, jnp.int32(_R2), 11)
    for r, v in enumerate([b1lo, b1hi, cblo, cbhi]):
        out_ref[r:r + 1, :] = jnp.full((1, 128), v, jnp.int32)
    for r in range(4, 8):
        out_ref[r:r + 1, :] = jnp.zeros((1, 128), jnp.int32)


def _tc2_kernel(h_ref, params_ref, out_ref):
    g_lo = jnp.sum(h_ref[0:32], axis=0, keepdims=True)
    g_hi = jnp.sum(h_ref[32:64], axis=0, keepdims=True)
    iota = lax.broadcasted_iota(jnp.int32, (1, _NB2), 1)
    b1lo = params_ref[0, 0]
    b1hi = params_ref[1, 0]
    cblo = params_ref[2, 0]
    cbhi = params_ref[3, 0]
    b2lo, wlo = _bucket_search(g_lo, iota, jnp.int32(_R1) - cblo, 11)
    b2hi, whi = _bucket_search(g_hi, iota, jnp.int32(_R2) - cbhi, 11)
    cb2lo = cblo + wlo
    cb2hi = cbhi + whi
    tslo = lax.bitwise_xor(
        lax.shift_left((b1lo << 11 | b2lo) + 1, 10), jnp.int32(_INT_MIN))
    tshi = lax.bitwise_xor(
        lax.shift_left((b1hi << 11 | b2hi) + 1, 10), jnp.int32(_INT_MIN))
    for r, v in enumerate([b1lo, b1hi, b2lo, b2hi, tslo, tshi, cb2lo, cb2hi]):
        out_ref[r:r + 1, :] = jnp.full((1, 128), v, jnp.int32)


def _tc3_kernel(h_ref, minab_ref, params_ref, ema_ref, out0_ref, out1_ref):
    iota = lax.broadcasted_iota(jnp.int32, (1, _NB3), 1)

    def resolve(rows, rank, frac, b1, b2, cb2, mrow):
        g = jnp.sum(h_ref[rows], axis=0, keepdims=True)
        b3, below = _bucket_search(g, iota, rank - cb2, 10)
        c_at = jnp.sum(jnp.where(iota == b3, g, 0))
        cnt_le = cb2 + below + c_at
        base = lax.bitwise_xor(
            lax.shift_left(b1 << 11 | b2, 10), jnp.int32(_INT_MIN))
        k1 = base | b3
        j2 = jnp.min(jnp.where((iota > b3) & (g > 0), iota, jnp.int32(_NB3)))
        k2_far = jnp.min(minab_ref[mrow:mrow + 1, :])
        k2 = jnp.where(cnt_le >= rank + 2, k1,
                       jnp.where(j2 < _NB3, base | j2, k2_far))
        v1 = _skey_to_float(k1)
        v2 = _skey_to_float(k2)
        return v1 + jnp.float32(frac) * (v2 - v1)

    q_lo = resolve(slice(0, 32), jnp.int32(_R1), _F1, params_ref[0, 0],
                   params_ref[2, 0], params_ref[6, 0], 0)
    q_hi = resolve(slice(32, 64), jnp.int32(_R2), _F2, params_ref[1, 0],
                   params_ref[3, 0], params_ref[7, 0], 1)
    new0 = jnp.float32(_ALPHA) * q_lo + jnp.float32(1 - _ALPHA) * ema_ref[0]
    new1 = jnp.float32(_ALPHA) * q_hi + jnp.float32(1 - _ALPHA) * ema_ref[1]
    out0_ref[...] = jnp.full((1, 128), new0, jnp.float32)
    out1_ref[...] = jnp.full((1, 128), jnp.maximum(new1 - new0, 1.0),
                             jnp.float32)


def kernel(x, ema_vals):
    xf = lax.bitcast_convert_type(x, jnp.int32).reshape(-1)
    mesh = plsc.VectorSubcoreMesh(core_axis_name="c", subcore_axis_name="s",
                                  num_cores=2, num_subcores=16)
    i32 = jnp.int32

    h1 = pl.kernel(
        _sc1_body,
        out_type=jax.ShapeDtypeStruct((_NW, _NB1), i32),
        mesh=mesh,
        compiler_params=pltpu.CompilerParams(needs_layout_passes=False),
        scratch_types=[pltpu.VMEM((_CHUNK,), i32),
                       pltpu.VMEM((16 * _NB1,), i32),
                       pltpu.VMEM((_NB1,), i32)],
    )(xf)

    params1 = pl.pallas_call(
        _tc1_kernel,
        out_shape=jax.ShapeDtypeStruct((8, 128), i32),
        in_specs=[pl.BlockSpec(memory_space=pltpu.VMEM)],
        out_specs=pl.BlockSpec(memory_space=pltpu.VMEM),
    )(h1)

    h2 = pl.kernel(
        _sc2_body,
        out_type=jax.ShapeDtypeStruct((2, _NW, _NB2), i32),
        mesh=mesh,
        compiler_params=pltpu.CompilerParams(needs_layout_passes=False),
        scratch_types=[pltpu.VMEM((_CHUNK,), i32),
                       pltpu.VMEM((2 * 16 * _NB2,), i32),
                       pltpu.VMEM((2 * _NB2,), i32),
                       pltpu.VMEM((16,), i32),
                       pltpu.VMEM((16,), i32)],
    )(xf, params1.reshape(-1))

    params2 = pl.pallas_call(
        _tc2_kernel,
        out_shape=jax.ShapeDtypeStruct((8, 128), i32),
        in_specs=[pl.BlockSpec(memory_space=pltpu.VMEM),
                  pl.BlockSpec(memory_space=pltpu.SMEM)],
        out_specs=pl.BlockSpec(memory_space=pltpu.VMEM),
    )(h2.reshape(2 * _NW, _NB2), params1)

    h3, minab = pl.kernel(
        _sc3_body,
        out_type=(jax.ShapeDtypeStruct((2, _NW, _NB3), i32),
                  jax.ShapeDtypeStruct((2 * _NW * 16,), i32)),
        mesh=mesh,
        compiler_params=pltpu.CompilerParams(needs_layout_passes=False),
        scratch_types=[pltpu.VMEM((_CHUNK,), i32),
                       pltpu.VMEM((2 * 16 * _NB3,), i32),
                       pltpu.VMEM((2 * _NB3,), i32),
                       pltpu.VMEM((6 * 16,), i32),
                       pltpu.VMEM((2 * 16,), i32)],
    )(xf, params2.reshape(-1))

    out0, out1 = pl.pallas_call(
        _tc3_kernel,
        out_shape=(jax.ShapeDtypeStruct((1, 128), jnp.float32),
                   jax.ShapeDtypeStruct((1, 128), jnp.float32)),
        in_specs=[pl.BlockSpec(memory_space=pltpu.VMEM),
                  pl.BlockSpec(memory_space=pltpu.VMEM),
                  pl.BlockSpec(memory_space=pltpu.SMEM),
                  pl.BlockSpec(memory_space=pltpu.SMEM)],
        out_specs=(pl.BlockSpec(memory_space=pltpu.VMEM),
                   pl.BlockSpec(memory_space=pltpu.VMEM)),
    )(h3.reshape(2 * _NW, _NB3), minab.reshape(2, _NW * 16), params2,
      ema_vals)
    return out0[0, 0], out1[0, 0]


# speedup vs baseline: 1.6594x; 1.1462x over previous
import jax
import jax.numpy as jnp
from jax import lax
from jax.experimental import pallas as pl
from jax.experimental.pallas import tpu as pltpu
from jax.experimental.pallas import tpu_sc as plsc

_INT_MIN = -2147483648

_N = 1048576
_NW = 32
_CHUNK = _N // _NW
_VPT = _CHUNK // 16
_NB1 = 4096
_NB2 = 1024
_NB3 = 1024
_NS1 = _NB1 + 1
_NS2 = _NB2 + 1
_NS3 = _NB3 + 1
_SZ1 = 65664
_SZ2 = 16512
_SZ3 = 16512
_R1 = int(0.05 * (_N - 1))
_F1 = 0.05 * (_N - 1) - _R1
_R2 = int(0.95 * (_N - 1))
_F2 = 0.95 * (_N - 1) - _R2
_RANKS = (_R1, _R1 + 1, _R2, _R2 + 1)
_ALPHA = 0.01


def _sc_skey(b):
    return jnp.where(b >= 0, b, jnp.int32(_INT_MIN) - b)


def _skey_to_float(k):
    return lax.bitcast_convert_type(
        jnp.where(k >= 0, k, jnp.int32(_INT_MIN) - k), jnp.float32)


def _b1_of(sk):
    return lax.bitwise_xor(
        lax.bitwise_and(lax.shift_right_arithmetic(sk, 20), jnp.int32(0xFFF)),
        jnp.int32(0x800))


def _b2_of(sk):
    return lax.bitwise_and(lax.shift_right_arithmetic(sk, 10), jnp.int32(0x3FF))


def _sc_wid():
    return lax.axis_index("s") * 2 + lax.axis_index("c")


def _zero_vmem(ref, nwords):
    z = jnp.zeros((16,), jnp.int32)

    @plsc.parallel_loop(0, nwords // 16, unroll=8)
    def _(i):
        ref[pl.ds(i * 16, 16)] = z


def _lane_reduce(hist_v, g_v, nb, stride):
    @plsc.parallel_loop(0, nb // 16, unroll=2)
    def _(j):
        acc = hist_v[pl.ds(j * 16, 16)]
        for l in range(1, 16):
            acc = acc + hist_v[pl.ds(l * stride + j * 16, 16)]
        g_v[pl.ds(j * 16, 16)] = acc


def _sc1_body(x_hbm, out_hbm, data_v, hist_v, g_v):
    wid = _sc_wid()
    pltpu.sync_copy(x_hbm.at[pl.ds(wid * _CHUNK, _CHUNK)], data_v)
    _zero_vmem(hist_v, _SZ1)
    laneoff = jnp.arange(16, dtype=jnp.int32) * _NS1
    ones = jnp.ones((16,), jnp.int32)

    @plsc.parallel_loop(0, _VPT, unroll=8)
    def _(i):
        sk = _sc_skey(data_v[pl.ds(i * 16, 16)])
        plsc.addupdate_scatter(hist_v, [laneoff + _b1_of(sk)], ones)

    _lane_reduce(hist_v, g_v, _NB1, _NS1)
    pltpu.sync_copy(g_v, out_hbm.at[wid])


def _sc2_body(x_hbm, params_hbm, out_hbm, data_v, hist_v, g_v, p_v):
    wid = _sc_wid()
    for c in range(4):
        pltpu.sync_copy(params_hbm.at[pl.ds(c * 128, 16)],
                        p_v.at[pl.ds(c * 16, 16)])
    pltpu.sync_copy(x_hbm.at[pl.ds(wid * _CHUNK, _CHUNK)], data_v)
    _zero_vmem(hist_v, 4 * _SZ2)
    laneoff = jnp.arange(16, dtype=jnp.int32) * _NS2
    ones = jnp.ones((16,), jnp.int32)
    pc = [p_v[pl.ds(c * 16, 16)] for c in range(4)]

    @plsc.parallel_loop(0, _VPT, unroll=8)
    def _(i):
        sk = _sc_skey(data_v[pl.ds(i * 16, 16)])
        b1 = _b1_of(sk)
        idx = laneoff + _b2_of(sk)
        for c in range(4):
            plsc.addupdate_scatter(hist_v, [idx + c * _SZ2], ones,
                                   mask=b1 == pc[c])

    for c in range(4):
        _lane_reduce(hist_v.at[pl.ds(c * _SZ2, _SZ2)],
                     g_v.at[pl.ds(c * _NB2, _NB2)], _NB2, _NS2)
        pltpu.sync_copy(g_v.at[pl.ds(c * _NB2, _NB2)], out_hbm.at[c, wid])


def _sc3_body(x_hbm, params_hbm, out_hbm, data_v, hist_v, g_v, p_v):
    wid = _sc_wid()
    for c in range(8):
        pltpu.sync_copy(params_hbm.at[pl.ds(c * 128, 16)],
                        p_v.at[pl.ds(c * 16, 16)])
    pltpu.sync_copy(x_hbm.at[pl.ds(wid * _CHUNK, _CHUNK)], data_v)
    _zero_vmem(hist_v, 4 * _SZ3)
    laneoff = jnp.arange(16, dtype=jnp.int32) * _NS3
    ones = jnp.ones((16,), jnp.int32)
    pc = [p_v[pl.ds(c * 16, 16)] for c in range(4)]
    qc = [p_v[pl.ds((4 + c) * 16, 16)] for c in range(4)]

    @plsc.parallel_loop(0, _VPT, unroll=8)
    def _(i):
        sk = _sc_skey(data_v[pl.ds(i * 16, 16)])
        b1 = _b1_of(sk)
        b2 = _b2_of(sk)
        idx = laneoff + lax.bitwise_and(sk, jnp.int32(0x3FF))
        for c in range(4):
            plsc.addupdate_scatter(hist_v, [idx + c * _SZ3], ones,
                                   mask=(b1 == pc[c]) & (b2 == qc[c]))

    for c in range(4):
        _lane_reduce(hist_v.at[pl.ds(c * _SZ3, _SZ3)],
                     g_v.at[pl.ds(c * _NB3, _NB3)], _NB3, _NS3)
        pltpu.sync_copy(g_v.at[pl.ds(c * _NB3, _NB3)], out_hbm.at[c, wid])


def _bucket_search(g, iota, rank, nbits):
    b = jnp.int32(0)
    for k in range(nbits - 1, -1, -1):
        cand = b + jnp.int32(1 << k)
        s = jnp.sum(jnp.where(iota < cand, g, 0))
        b = jnp.where(s <= rank, cand, b)
    below = jnp.sum(jnp.where(iota < b, g, 0))
    return b, below


def _tc1_kernel(h_ref, out_ref):
    g = jnp.sum(h_ref[...], axis=0, keepdims=True)
    iota = lax.broadcasted_iota(jnp.int32, (1, _NB1), 1)
    for c, rank in enumerate(_RANKS):
        b1, cb = _bucket_search(g, iota, jnp.int32(rank), 12)
        out_ref[c:c + 1, :] = jnp.full((1, 128), b1, jnp.int32)
        out_ref[c + 4:c + 5, :] = jnp.full((1, 128), cb, jnp.int32)


def _tc2_kernel(h_ref, params_ref, out_ref):
    iota = lax.broadcasted_iota(jnp.int32, (1, _NB2), 1)
    for c, rank in enumerate(_RANKS):
        g = jnp.sum(h_ref[c * _NW:(c + 1) * _NW], axis=0, keepdims=True)
        cb = params_ref[c + 4, 0]
        b2, below = _bucket_search(g, iota, jnp.int32(rank) - cb, 10)
        out_ref[c:c + 1, :] = jnp.full((1, 128), params_ref[c, 0], jnp.int32)
        out_ref[c + 4:c + 5, :] = jnp.full((1, 128), b2, jnp.int32)
        out_ref[c + 8:c + 9, :] = jnp.full((1, 128), cb + below, jnp.int32)
    for r in range(12, 16):
        out_ref[r:r + 1, :] = jnp.zeros((1, 128), jnp.int32)


def _tc3_kernel(h_ref, params_ref, ema_ref, out0_ref, out1_ref):
    iota = lax.broadcasted_iota(jnp.int32, (1, _NB3), 1)
    vals = []
    for c, rank in enumerate(_RANKS):
        g = jnp.sum(h_ref[c * _NW:(c + 1) * _NW], axis=0, keepdims=True)
        b1 = params_ref[c, 0]
        b2 = params_ref[c + 4, 0]
        cb2 = params_ref[c + 8, 0]
        b3, _ = _bucket_search(g, iota, jnp.int32(rank) - cb2, 10)
        k = (lax.bitwise_xor(b1, jnp.int32(0x800)) << 20) | (b2 << 10) | b3
        vals.append(_skey_to_float(k))
    q_lo = vals[0] + jnp.float32(_F1) * (vals[1] - vals[0])
    q_hi = vals[2] + jnp.float32(_F2) * (vals[3] - vals[2])
    new0 = jnp.float32(_ALPHA) * q_lo + jnp.float32(1 - _ALPHA) * ema_ref[0]
    new1 = jnp.float32(_ALPHA) * q_hi + jnp.float32(1 - _ALPHA) * ema_ref[1]
    out0_ref[...] = jnp.full((1, 128), new0, jnp.float32)
    out1_ref[...] = jnp.full((1, 128), jnp.maximum(new1 - new0, 1.0),
                             jnp.float32)


def kernel(x, ema_vals):
    xf = lax.bitcast_convert_type(x, jnp.int32).reshape(-1)
    mesh = plsc.VectorSubcoreMesh(core_axis_name="c", subcore_axis_name="s",
                                  num_cores=2, num_subcores=16)
    i32 = jnp.int32
    sc_params = pltpu.CompilerParams(needs_layout_passes=False)

    h1 = pl.kernel(
        _sc1_body,
        out_type=jax.ShapeDtypeStruct((_NW, _NB1), i32),
        mesh=mesh,
        compiler_params=sc_params,
        scratch_types=[pltpu.VMEM((_CHUNK,), i32),
                       pltpu.VMEM((_SZ1,), i32),
                       pltpu.VMEM((_NB1,), i32)],
    )(xf)

    params1 = pl.pallas_call(
        _tc1_kernel,
        out_shape=jax.ShapeDtypeStruct((8, 128), i32),
        in_specs=[pl.BlockSpec(memory_space=pltpu.VMEM)],
        out_specs=pl.BlockSpec(memory_space=pltpu.VMEM),
    )(h1)

    h2 = pl.kernel(
        _sc2_body,
        out_type=jax.ShapeDtypeStruct((4, _NW, _NB2), i32),
        mesh=mesh,
        compiler_params=sc_params,
        scratch_types=[pltpu.VMEM((_CHUNK,), i32),
                       pltpu.VMEM((4 * _SZ2,), i32),
                       pltpu.VMEM((4 * _NB2,), i32),
                       pltpu.VMEM((4 * 16,), i32)],
    )(xf, params1.reshape(-1))

    params2 = pl.pallas_call(
        _tc2_kernel,
        out_shape=jax.ShapeDtypeStruct((16, 128), i32),
        in_specs=[pl.BlockSpec(memory_space=pltpu.VMEM),
                  pl.BlockSpec(memory_space=pltpu.SMEM)],
        out_specs=pl.BlockSpec(memory_space=pltpu.VMEM),
    )(h2.reshape(4 * _NW, _NB2), params1)

    h3 = pl.kernel(
        _sc3_body,
        out_type=jax.ShapeDtypeStruct((4, _NW, _NB3), i32),
        mesh=mesh,
        compiler_params=sc_params,
        scratch_types=[pltpu.VMEM((_CHUNK,), i32),
                       pltpu.VMEM((4 * _SZ3,), i32),
                       pltpu.VMEM((4 * _NB3,), i32),
                       pltpu.VMEM((8 * 16,), i32)],
    )(xf, params2.reshape(-1))

    out0, out1 = pl.pallas_call(
        _tc3_kernel,
        out_shape=(jax.ShapeDtypeStruct((1, 128), jnp.float32),
                   jax.ShapeDtypeStruct((1, 128), jnp.float32)),
        in_specs=[pl.BlockSpec(memory_space=pltpu.VMEM),
                  pl.BlockSpec(memory_space=pltpu.SMEM),
                  pl.BlockSpec(memory_space=pltpu.SMEM)],
        out_specs=(pl.BlockSpec(memory_space=pltpu.VMEM),
                   pl.BlockSpec(memory_space=pltpu.VMEM)),
    )(h3.reshape(4 * _NW, _NB3), params2, ema_vals)
    return out0[0, 0], out1[0, 0]


# speedup vs baseline: 1.8878x; 1.1376x over previous
import jax
import jax.numpy as jnp
from jax import lax
from jax.experimental import pallas as pl
from jax.experimental.pallas import tpu as pltpu
from jax.experimental.pallas import tpu_sc as plsc

_INT_MIN = -2147483648

_N = 1048576
_NW = 32
_CHUNK = _N // _NW
_VPT = _CHUNK // 16
_NB1 = 4096
_NB2 = 1024
_NB3 = 1024
_NS1 = _NB1 + 1
_NS2 = _NB2 + 1
_NS3 = _NB3 + 1
_SZ1 = 65664
_SZ2 = 16512
_SZ3 = 16512
_R1 = int(0.05 * (_N - 1))
_F1 = 0.05 * (_N - 1) - _R1
_R2 = int(0.95 * (_N - 1))
_F2 = 0.95 * (_N - 1) - _R2
_RANKS = (_R1, _R1 + 1, _R2, _R2 + 1)
_ALPHA = 0.01


def _sc_skey(b):
    return jnp.where(b >= 0, b, jnp.int32(_INT_MIN) - b)


def _skey_to_float(k):
    return lax.bitcast_convert_type(
        jnp.where(k >= 0, k, jnp.int32(_INT_MIN) - k), jnp.float32)


def _b1_of(sk):
    return lax.bitwise_xor(
        lax.bitwise_and(lax.shift_right_arithmetic(sk, 20), jnp.int32(0xFFF)),
        jnp.int32(0x800))


def _b2_of(sk):
    return lax.bitwise_and(lax.shift_right_arithmetic(sk, 10), jnp.int32(0x3FF))


def _sc_wid():
    return lax.axis_index("s") * 2 + lax.axis_index("c")


def _zero_vmem(ref, nwords):
    z = jnp.zeros((16,), jnp.int32)

    @plsc.parallel_loop(0, nwords // 16, unroll=8)
    def _(i):
        ref[pl.ds(i * 16, 16)] = z


def _lane_reduce(hist_v, g_v, nb, stride):
    @plsc.parallel_loop(0, nb // 16, unroll=2)
    def _(j):
        acc = hist_v[pl.ds(j * 16, 16)]
        for l in range(1, 16):
            acc = acc + hist_v[pl.ds(l * stride + j * 16, 16)]
        g_v[pl.ds(j * 16, 16)] = acc


def _sc1_body(x_hbm, out_hbm, data_v, hist_v, g_v):
    wid = _sc_wid()
    pltpu.sync_copy(x_hbm.at[pl.ds(wid * _CHUNK, _CHUNK)], data_v)
    _zero_vmem(hist_v, _SZ1)
    laneoff = jnp.arange(16, dtype=jnp.int32) * _NS1
    ones = jnp.ones((16,), jnp.int32)

    @plsc.parallel_loop(0, _VPT, unroll=8)
    def _(i):
        sk = _sc_skey(data_v[pl.ds(i * 16, 16)])
        plsc.addupdate_scatter(hist_v, [laneoff + _b1_of(sk)], ones)

    _lane_reduce(hist_v, g_v, _NB1, _NS1)
    pltpu.sync_copy(g_v, out_hbm.at[wid])


def _sc2_body(x_hbm, params_hbm, out_hbm, data_v, hist_v, g_v, p_v):
    wid = _sc_wid()
    for c in range(4):
        pltpu.sync_copy(params_hbm.at[pl.ds(c * 128, 16)],
                        p_v.at[pl.ds(c * 16, 16)])
    pltpu.sync_copy(x_hbm.at[pl.ds(wid * _CHUNK, _CHUNK)], data_v)
    _zero_vmem(hist_v, 4 * _SZ2)
    laneoff = jnp.arange(16, dtype=jnp.int32) * _NS2
    ones = jnp.ones((16,), jnp.int32)
    pc = [p_v[pl.ds(c * 16, 16)] for c in range(4)]
    dump = jnp.full((16,), 4, jnp.int32)

    @plsc.parallel_loop(0, _VPT, unroll=8)
    def _(i):
        sk = _sc_skey(data_v[pl.ds(i * 16, 16)])
        b1 = _b1_of(sk)
        t = dump
        for c in range(3, -1, -1):
            t = jnp.where(b1 == pc[c], jnp.int32(c), t)
        plsc.addupdate_scatter(
            hist_v, [laneoff + _b2_of(sk) + t * _SZ2], ones)

    for c in range(4):
        _lane_reduce(hist_v.at[pl.ds(c * _SZ2, _SZ2)],
                     g_v.at[pl.ds(c * _NB2, _NB2)], _NB2, _NS2)
        pltpu.sync_copy(g_v.at[pl.ds(c * _NB2, _NB2)], out_hbm.at[c, wid])


def _sc3_body(x_hbm, params_hbm, out_hbm, data_v, hist_v, g_v, p_v):
    wid = _sc_wid()
    for c in range(8):
        pltpu.sync_copy(params_hbm.at[pl.ds(c * 128, 16)],
                        p_v.at[pl.ds(c * 16, 16)])
    pltpu.sync_copy(x_hbm.at[pl.ds(wid * _CHUNK, _CHUNK)], data_v)
    _zero_vmem(hist_v, 4 * _SZ3)
    laneoff = jnp.arange(16, dtype=jnp.int32) * _NS3
    ones = jnp.ones((16,), jnp.int32)
    pc = [p_v[pl.ds(c * 16, 16)] for c in range(4)]
    qc = [p_v[pl.ds((4 + c) * 16, 16)] for c in range(4)]
    dump = jnp.full((16,), 4, jnp.int32)

    @plsc.parallel_loop(0, _VPT, unroll=8)
    def _(i):
        sk = _sc_skey(data_v[pl.ds(i * 16, 16)])
        b1 = _b1_of(sk)
        b2 = _b2_of(sk)
        t = dump
        for c in range(3, -1, -1):
            t = jnp.where((b1 == pc[c]) & (b2 == qc[c]), jnp.int32(c), t)
        plsc.addupdate_scatter(
            hist_v, [laneoff + lax.bitwise_and(sk, jnp.int32(0x3FF))
                     + t * _SZ3], ones)

    for c in range(4):
        _lane_reduce(hist_v.at[pl.ds(c * _SZ3, _SZ3)],
                     g_v.at[pl.ds(c * _NB3, _NB3)], _NB3, _NS3)
        pltpu.sync_copy(g_v.at[pl.ds(c * _NB3, _NB3)], out_hbm.at[c, wid])


def _bucket_search(g, iota, rank, nbits):
    b = jnp.int32(0)
    for k in range(nbits - 1, -1, -1):
        cand = b + jnp.int32(1 << k)
        s = jnp.sum(jnp.where(iota < cand, g, 0))
        b = jnp.where(s <= rank, cand, b)
    below = jnp.sum(jnp.where(iota < b, g, 0))
    return b, below


def _tc1_kernel(h_ref, out_ref):
    g = jnp.sum(h_ref[...], axis=0, keepdims=True)
    iota = lax.broadcasted_iota(jnp.int32, (1, _NB1), 1)
    for c, rank in enumerate(_RANKS):
        b1, cb = _bucket_search(g, iota, jnp.int32(rank), 12)
        out_ref[c:c + 1, :] = jnp.full((1, 128), b1, jnp.int32)
        out_ref[c + 4:c + 5, :] = jnp.full((1, 128), cb, jnp.int32)


def _tc2_kernel(h_ref, params_ref, out_ref):
    iota = lax.broadcasted_iota(jnp.int32, (1, _NB2), 1)
    gs = [jnp.sum(h_ref[c * _NW:(c + 1) * _NW], axis=0, keepdims=True)
          for c in range(4)]
    for c, rank in enumerate(_RANKS):
        g = gs[c]
        for j in range(c - 1, -1, -1):
            g = jnp.where(params_ref[c, 0] == params_ref[j, 0], gs[j], g)
        cb = params_ref[c + 4, 0]
        b2, below = _bucket_search(g, iota, jnp.int32(rank) - cb, 10)
        out_ref[c:c + 1, :] = jnp.full((1, 128), params_ref[c, 0], jnp.int32)
        out_ref[c + 4:c + 5, :] = jnp.full((1, 128), b2, jnp.int32)
        out_ref[c + 8:c + 9, :] = jnp.full((1, 128), cb + below, jnp.int32)
    for r in range(12, 16):
        out_ref[r:r + 1, :] = jnp.zeros((1, 128), jnp.int32)


def _tc3_kernel(h_ref, params_ref, ema_ref, out0_ref, out1_ref):
    iota = lax.broadcasted_iota(jnp.int32, (1, _NB3), 1)
    gs = [jnp.sum(h_ref[c * _NW:(c + 1) * _NW], axis=0, keepdims=True)
          for c in range(4)]
    vals = []
    for c, rank in enumerate(_RANKS):
        g = gs[c]
        for j in range(c - 1, -1, -1):
            same = ((params_ref[c, 0] == params_ref[j, 0])
                    & (params_ref[c + 4, 0] == params_ref[j + 4, 0]))
            g = jnp.where(same, gs[j], g)
        b1 = params_ref[c, 0]
        b2 = params_ref[c + 4, 0]
        cb2 = params_ref[c + 8, 0]
        b3, _ = _bucket_search(g, iota, jnp.int32(rank) - cb2, 10)
        k = (lax.bitwise_xor(b1, jnp.int32(0x800)) << 20) | (b2 << 10) | b3
        vals.append(_skey_to_float(k))
    q_lo = vals[0] + jnp.float32(_F1) * (vals[1] - vals[0])
    q_hi = vals[2] + jnp.float32(_F2) * (vals[3] - vals[2])
    new0 = jnp.float32(_ALPHA) * q_lo + jnp.float32(1 - _ALPHA) * ema_ref[0]
    new1 = jnp.float32(_ALPHA) * q_hi + jnp.float32(1 - _ALPHA) * ema_ref[1]
    out0_ref[...] = jnp.full((1, 128), new0, jnp.float32)
    out1_ref[...] = jnp.full((1, 128), jnp.maximum(new1 - new0, 1.0),
                             jnp.float32)


def kernel(x, ema_vals):
    xf = lax.bitcast_convert_type(x, jnp.int32).reshape(-1)
    mesh = plsc.VectorSubcoreMesh(core_axis_name="c", subcore_axis_name="s",
                                  num_cores=2, num_subcores=16)
    i32 = jnp.int32
    sc_params = pltpu.CompilerParams(needs_layout_passes=False)

    h1 = pl.kernel(
        _sc1_body,
        out_type=jax.ShapeDtypeStruct((_NW, _NB1), i32),
        mesh=mesh,
        compiler_params=sc_params,
        scratch_types=[pltpu.VMEM((_CHUNK,), i32),
                       pltpu.VMEM((_SZ1,), i32),
                       pltpu.VMEM((_NB1,), i32)],
    )(xf)

    params1 = pl.pallas_call(
        _tc1_kernel,
        out_shape=jax.ShapeDtypeStruct((8, 128), i32),
        in_specs=[pl.BlockSpec(memory_space=pltpu.VMEM)],
        out_specs=pl.BlockSpec(memory_space=pltpu.VMEM),
    )(h1)

    h2 = pl.kernel(
        _sc2_body,
        out_type=jax.ShapeDtypeStruct((4, _NW, _NB2), i32),
        mesh=mesh,
        compiler_params=sc_params,
        scratch_types=[pltpu.VMEM((_CHUNK,), i32),
                       pltpu.VMEM((5 * _SZ2,), i32),
                       pltpu.VMEM((4 * _NB2,), i32),
                       pltpu.VMEM((4 * 16,), i32)],
    )(xf, params1.reshape(-1))

    params2 = pl.pallas_call(
        _tc2_kernel,
        out_shape=jax.ShapeDtypeStruct((16, 128), i32),
        in_specs=[pl.BlockSpec(memory_space=pltpu.VMEM),
                  pl.BlockSpec(memory_space=pltpu.SMEM)],
        out_specs=pl.BlockSpec(memory_space=pltpu.VMEM),
    )(h2.reshape(4 * _NW, _NB2), params1)

    h3 = pl.kernel(
        _sc3_body,
        out_type=jax.ShapeDtypeStruct((4, _NW, _NB3), i32),
        mesh=mesh,
        compiler_params=sc_params,
        scratch_types=[pltpu.VMEM((_CHUNK,), i32),
                       pltpu.VMEM((5 * _SZ3,), i32),
                       pltpu.VMEM((4 * _NB3,), i32),
                       pltpu.VMEM((8 * 16,), i32)],
    )(xf, params2.reshape(-1))

    out0, out1 = pl.pallas_call(
        _tc3_kernel,
        out_shape=(jax.ShapeDtypeStruct((1, 128), jnp.float32),
                   jax.ShapeDtypeStruct((1, 128), jnp.float32)),
        in_specs=[pl.BlockSpec(memory_space=pltpu.VMEM),
                  pl.BlockSpec(memory_space=pltpu.SMEM),
                  pl.BlockSpec(memory_space=pltpu.SMEM)],
        out_specs=(pl.BlockSpec(memory_space=pltpu.VMEM),
                   pl.BlockSpec(memory_space=pltpu.VMEM)),
    )(h3.reshape(4 * _NW, _NB3), params2, ema_vals)
    return out0[0, 0], out1[0, 0]
